# speedup vs baseline: 4.2681x; 4.2681x over previous
import jax
import jax.numpy as jnp
import numpy as np
from jax import lax
from jax.experimental import pallas as pl
from jax.experimental.pallas import tpu as pltpu
from jax.experimental.pallas import tpu_sc as plsc

MAX_LEN = 200
EMBED_DIM = 128
BATCH = 4096

NUM_CORES = 2
NUM_SUBCORES = 16
NUM_WORKERS = NUM_CORES * NUM_SUBCORES
SEQS_PER_WORKER = BATCH // NUM_WORKERS
LANES = 16
VECS_PER_ROW = EMBED_DIM // LANES


def _make_pe_np():
    pos = np.arange(MAX_LEN, dtype=np.float64)[:, None]
    j = np.arange(EMBED_DIM, dtype=np.float64)[None, :]
    angle = pos / (10000.0 ** (j / float(EMBED_DIM)))
    pe = np.where((np.arange(EMBED_DIM)[None, :] % 2) == 0, np.sin(angle), np.cos(angle))
    return pe.astype(np.float32)


_PE = _make_pe_np()


def _sc_body(x_hbm, table_hbm, pe_hbm, out_hbm, idx_v, rows_v, pe_v, sem):
    wid = lax.axis_index("s") * NUM_CORES + lax.axis_index("c")
    seq0 = wid * SEQS_PER_WORKER

    pltpu.sync_copy(pe_hbm, pe_v)

    def per_seq(i, carry):
        row0 = (seq0 + i) * MAX_LEN
        pltpu.sync_copy(x_hbm.at[pl.ds(row0, MAX_LEN)], idx_v)
        pltpu.async_copy(table_hbm.at[idx_v], rows_v, sem).wait()

        def per_row(r, c2):
            for c in range(VECS_PER_ROW):
                sl = pl.ds(c * LANES, LANES)
                rows_v[r, sl] = rows_v[r, sl] + pe_v[r, sl]
            return c2

        lax.fori_loop(0, MAX_LEN, per_row, 0)
        pltpu.sync_copy(rows_v, out_hbm.at[pl.ds(row0, MAX_LEN)])
        return carry

    lax.fori_loop(0, SEQS_PER_WORKER, per_seq, 0)


@jax.jit
def _pos_embed(x_flat, table, pe):
    mesh = plsc.VectorSubcoreMesh(core_axis_name="c", subcore_axis_name="s")
    return pl.kernel(
        _sc_body,
        out_type=jax.ShapeDtypeStruct((BATCH * MAX_LEN, EMBED_DIM), jnp.float32),
        mesh=mesh,
        scratch_types=[
            pltpu.VMEM((MAX_LEN,), jnp.int32),
            pltpu.VMEM((MAX_LEN, EMBED_DIM), jnp.float32),
            pltpu.VMEM((MAX_LEN, EMBED_DIM), jnp.float32),
            pltpu.SemaphoreType.DMA,
        ],
    )(x_flat, table, pe)


def kernel(x, embed_weight):
    x_flat = x.reshape(-1).astype(jnp.int32)
    pe = jnp.asarray(_PE)
    out = _pos_embed(x_flat, embed_weight, pe)
    return out.reshape(BATCH, MAX_LEN, EMBED_DIM)


# speedup vs baseline: 9.0492x; 2.1202x over previous
import jax
import jax.numpy as jnp
import numpy as np
from jax import lax
from jax.experimental import pallas as pl
from jax.experimental.pallas import tpu as pltpu
from jax.experimental.pallas import tpu_sc as plsc

MAX_LEN = 200
EMBED_DIM = 128
BATCH = 4096

NUM_CORES = 2
NUM_SUBCORES = 16
NUM_WORKERS = NUM_CORES * NUM_SUBCORES
SEQS_PER_WORKER = BATCH // NUM_WORKERS
LANES = 16
VECS_PER_ROW = EMBED_DIM // LANES


def _make_pe_np():
    pos = np.arange(MAX_LEN, dtype=np.float64)[:, None]
    j = np.arange(EMBED_DIM, dtype=np.float64)[None, :]
    angle = pos / (10000.0 ** (j / float(EMBED_DIM)))
    pe = np.where((np.arange(EMBED_DIM)[None, :] % 2) == 0, np.sin(angle), np.cos(angle))
    return pe.astype(np.float32)


_PE = _make_pe_np()


def _sc_body(x_hbm, table_hbm, pe_hbm, out_hbm,
             idx0, idx1, rows0, rows1, o0, o1, pe_v,
             gsem0, gsem1, isem0, isem1, osem0, osem1):
    idx = (idx0, idx1)
    rows = (rows0, rows1)
    outb = (o0, o1)
    gsem = (gsem0, gsem1)
    isem = (isem0, isem1)
    osem = (osem0, osem1)

    wid = lax.axis_index("s") * NUM_CORES + lax.axis_index("c")
    seq0 = wid * SEQS_PER_WORKER

    pltpu.sync_copy(pe_hbm, pe_v)

    def idx_copy(j, b):
        row0 = (seq0 + j) * MAX_LEN
        return pltpu.make_async_copy(x_hbm.at[pl.ds(row0, MAX_LEN)], idx[b], isem[b])

    def gather(b):
        return pltpu.make_async_copy(table_hbm.at[idx[b]], rows[b], gsem[b])

    def store(j, b):
        row0 = (seq0 + j) * MAX_LEN
        return pltpu.make_async_copy(outb[b], out_hbm.at[pl.ds(row0, MAX_LEN)], osem[b])

    pltpu.sync_copy(x_hbm.at[pl.ds(seq0 * MAX_LEN, MAX_LEN)], idx0)
    pltpu.sync_copy(x_hbm.at[pl.ds((seq0 + 1) * MAX_LEN, MAX_LEN)], idx1)
    gather(0).start()
    gather(1).start()

    def pair(k, carry):
        for b in range(2):
            j = 2 * k + b
            gather(b).wait()

            @pl.when(k <= 62)
            def _():
                idx_copy(j + 2, b).start()

            @pl.when(k >= 1)
            def _():
                store(j - 2, b).wait()

            def per_row(r, c2):
                for c in range(VECS_PER_ROW):
                    sl = pl.ds(c * LANES, LANES)
                    outb[b][r, sl] = rows[b][r, sl] + pe_v[r, sl]
                return c2

            lax.fori_loop(0, MAX_LEN, per_row, 0)

            @pl.when(k <= 62)
            def _():
                idx_copy(j + 2, b).wait()
                gather(b).start()

            store(j, b).start()
        return carry

    lax.fori_loop(0, SEQS_PER_WORKER // 2, pair, 0)

    store(SEQS_PER_WORKER - 2, 0).wait()
    store(SEQS_PER_WORKER - 1, 1).wait()


@jax.jit
def _pos_embed(x_flat, table, pe):
    mesh = plsc.VectorSubcoreMesh(core_axis_name="c", subcore_axis_name="s")
    return pl.kernel(
        _sc_body,
        out_type=jax.ShapeDtypeStruct((BATCH * MAX_LEN, EMBED_DIM), jnp.float32),
        mesh=mesh,
        scratch_types=[
            pltpu.VMEM((MAX_LEN,), jnp.int32),
            pltpu.VMEM((MAX_LEN,), jnp.int32),
            pltpu.VMEM((MAX_LEN, EMBED_DIM), jnp.float32),
            pltpu.VMEM((MAX_LEN, EMBED_DIM), jnp.float32),
            pltpu.VMEM((MAX_LEN, EMBED_DIM), jnp.float32),
            pltpu.VMEM((MAX_LEN, EMBED_DIM), jnp.float32),
            pltpu.VMEM((MAX_LEN, EMBED_DIM), jnp.float32),
            pltpu.SemaphoreType.DMA,
            pltpu.SemaphoreType.DMA,
            pltpu.SemaphoreType.DMA,
            pltpu.SemaphoreType.DMA,
            pltpu.SemaphoreType.DMA,
            pltpu.SemaphoreType.DMA,
        ],
    )(x_flat, table, pe)


def kernel(x, embed_weight):
    x_flat = x.reshape(-1).astype(jnp.int32)
    pe = jnp.asarray(_PE)
    out = _pos_embed(x_flat, embed_weight, pe)
    return out.reshape(BATCH, MAX_LEN, EMBED_DIM)
